# baseline (speedup 1.0000x reference)
import functools

import jax
import jax.numpy as jnp
from jax import lax
from jax.experimental import pallas as pl
from jax.experimental.pallas import tpu as pltpu
from jax.experimental.pallas import tpu_sc as plsc

_EPS = 1e-5
_BPP = 2


def _argmin_loss_body(x_ref, w_ref, xs_ref, w2_ref, idx_ref, loss_ref,
                      q_ref=None):
    nb, T, C = x_ref.shape
    x = x_ref[...].reshape(nb * T, C)
    w = w_ref[...]
    K = w.shape[0]
    R = nb * T

    s = lax.dot_general(x, w, (((1,), (1,)), ((), ())),
                        preferred_element_type=jnp.float32)
    xs = xs_ref[...].reshape(R, 1)
    w2 = w2_ref[...]
    d2 = (xs + w2) - 2.0 * s

    dmin = jnp.min(d2, axis=1, keepdims=True)
    eqm = d2 == dmin
    kiota = lax.broadcasted_iota(jnp.int32, (R, K), 1).astype(jnp.float32)
    idx = jnp.min(jnp.where(eqm, kiota, float(K)), axis=1,
                  keepdims=True)
    p = pl.program_id(0)
    idx_ref[pl.ds(p * R, R)] = idx[:, 0].astype(jnp.int32)

    oh = kiota == idx
    ef = oh.astype(jnp.float32)
    w2_win = lax.dot_general(ef, w2, (((1,), (1,)), ((), ())),
                             precision=lax.Precision.HIGHEST,
                             preferred_element_type=jnp.float32)
    s_win = 0.5 * ((xs + w2_win) - dmin)
    xn = jnp.maximum(jnp.sqrt(xs), _EPS)
    wn = jnp.maximum(jnp.sqrt(w2_win), _EPS)
    row = (w2_win / (wn * wn) + xs / (xn * xn)
           - 2.0 * s_win / (wn * xn))
    row2 = row.reshape(nb, T)
    for j in range(nb):
        loss_ref[p * nb + j] = jnp.sum(row2[j]) / (T * C)

    if q_ref is not None:
        quant = lax.dot_general(ef, w, (((1,), (0,)), ((), ())),
                                precision=lax.Precision.HIGHEST,
                                preferred_element_type=jnp.float32)
        q_ref[...] = quant.reshape(nb, T, C)


def _argmin_and_loss(x, W, x2, w2, off, nb, with_quant=False):
    B, T, C = x.shape
    K = W.shape[0]
    grid = nb // _BPP
    boff = off // _BPP
    assert off % _BPP == 0
    out_specs = [
        pl.BlockSpec((nb * T,), lambda b: (0,)),
        pl.BlockSpec(memory_space=pltpu.SMEM),
    ]
    out_shape = [
        jax.ShapeDtypeStruct((nb * T,), jnp.int32),
        jax.ShapeDtypeStruct((nb,), jnp.float32),
    ]
    body = _argmin_loss_body
    if with_quant:
        out_specs.append(
            pl.BlockSpec((_BPP, T, C), lambda b, boff=boff: (b + boff, 0, 0)))
        out_shape.append(jax.ShapeDtypeStruct((B, T, C), jnp.float32))
        body = functools.partial(_argmin_loss_body)
    return pl.pallas_call(
        body,
        grid=(grid,),
        in_specs=[
            pl.BlockSpec((_BPP, T, C), lambda b, boff=boff: (b + boff, 0, 0)),
            pl.BlockSpec((K, C), lambda b: (0, 0)),
            pl.BlockSpec((_BPP, T, 1), lambda b, boff=boff: (b + boff, 0, 0)),
            pl.BlockSpec((1, K), lambda b: (0, 0)),
        ],
        out_specs=out_specs,
        out_shape=out_shape,
    )(x, W, x2, w2)


@functools.cache
def _make_sc_gather(V, D, B, OUT_ROWS):
    info = plsc.get_sparse_core_info()
    NC, NS = info.num_cores, info.num_subcores
    NW = NC * NS
    assert B % (8 * NW) == 0
    b_per_w = B // NW
    NCH = 3
    CH = b_per_w // NCH
    assert CH % 8 == 0
    mesh = plsc.VectorSubcoreMesh(core_axis_name="c", subcore_axis_name="s")

    @functools.partial(
        pl.kernel, mesh=mesh,
        out_type=jax.ShapeDtypeStruct((OUT_ROWS, D), jnp.float32),
        scratch_types=[
            pltpu.VMEM((b_per_w,), jnp.int32),
            pltpu.VMEM((NCH, CH, D), jnp.float32),
            [pltpu.SemaphoreType.DMA] * NCH,
            pltpu.SemaphoreType.DMA,
        ],
    )
    def gather(table_hbm, idx_hbm, out_hbm, idx_v, rows_v, gsems, wsem):
        wid = lax.axis_index("s") * NC + lax.axis_index("c")
        base = wid * b_per_w
        pltpu.sync_copy(idx_hbm.at[pl.ds(base, b_per_w)], idx_v)
        hs = [pltpu.async_copy(table_hbm.at[idx_v.at[pl.ds(c * CH, CH)]],
                               rows_v.at[c], gsems[c])
              for c in range(NCH)]
        ws = []
        for c in range(NCH):
            hs[c].wait()
            ws.append(pltpu.async_copy(
                rows_v.at[c], out_hbm.at[pl.ds(base + c * CH, CH)], wsem))
        for w in ws:
            w.wait()

    return gather


def kernel(x, W):
    B, T, C = x.shape
    K = W.shape[0]
    H = B // 2
    x2 = jnp.sum(x * x, axis=-1, keepdims=True)
    w2 = jnp.sum(W * W, axis=-1)[None, :]
    idx_a, loss_a = _argmin_and_loss(x, W, x2, w2, 0, H)
    quant_a = _make_sc_gather(K, C, H * T, H * T)(W, idx_a)
    idx_b, loss_b, quant_bf = _argmin_and_loss(x, W, x2, w2, H, B - H,
                                               with_quant=True)
    quant = lax.dynamic_update_slice(
        quant_bf.reshape(B * T, C), quant_a, (0, 0)).reshape(B, T, C)
    idx = jnp.concatenate([idx_a, idx_b]).reshape(B, T)
    loss = jnp.concatenate([loss_a, loss_b])
    return quant, loss, loss, idx


# speedup vs baseline: 1.1648x; 1.1648x over previous
import functools

import jax
import jax.numpy as jnp
from jax import lax
from jax.experimental import pallas as pl
from jax.experimental.pallas import tpu as pltpu
from jax.experimental.pallas import tpu_sc as plsc

_EPS = 1e-5
_BPP = 2


def _argmin_loss_body(x_ref, w_ref, xs_ref, w2_ref, idx_ref, loss_ref,
                      q_ref=None):
    nb, T, C = x_ref.shape
    x = x_ref[...].reshape(nb * T, C)
    w = w_ref[...]
    K = w.shape[0]
    R = nb * T

    s = lax.dot_general(x, w, (((1,), (1,)), ((), ())),
                        preferred_element_type=jnp.float32)
    xs = xs_ref[...].reshape(R, 1)
    w2 = w2_ref[...]
    d2 = (xs + w2) - 2.0 * s

    dmin = jnp.min(d2, axis=1, keepdims=True)
    eqm = d2 == dmin
    kiota = lax.broadcasted_iota(jnp.int32, (R, K), 1).astype(jnp.float32)
    idx = jnp.min(jnp.where(eqm, kiota, float(K)), axis=1,
                  keepdims=True)
    p = pl.program_id(0)
    idx_ref[pl.ds(p * R, R)] = idx[:, 0].astype(jnp.int32)

    oh = kiota == idx
    ef = oh.astype(jnp.float32)
    w2_win = lax.dot_general(ef, w2, (((1,), (1,)), ((), ())),
                             preferred_element_type=jnp.float32)
    s_win = 0.5 * ((xs + w2_win) - dmin)
    xn = jnp.maximum(jnp.sqrt(xs), _EPS)
    wn = jnp.maximum(jnp.sqrt(w2_win), _EPS)
    row = (w2_win / (wn * wn) + xs / (xn * xn)
           - 2.0 * s_win / (wn * xn))
    row2 = row.reshape(nb, T)
    for j in range(nb):
        loss_ref[p * nb + j] = jnp.sum(row2[j]) / (T * C)

    if q_ref is not None:
        quant = lax.dot_general(ef, w, (((1,), (0,)), ((), ())),
                                preferred_element_type=jnp.float32)
        q_ref[...] = quant.reshape(nb, T, C)


def _argmin_and_loss(x, W, x2, w2, off, nb, with_quant=False):
    B, T, C = x.shape
    K = W.shape[0]
    grid = nb // _BPP
    boff = off // _BPP
    assert off % _BPP == 0
    out_specs = [
        pl.BlockSpec((nb * T,), lambda b: (0,)),
        pl.BlockSpec(memory_space=pltpu.SMEM),
    ]
    out_shape = [
        jax.ShapeDtypeStruct((nb * T,), jnp.int32),
        jax.ShapeDtypeStruct((nb,), jnp.float32),
    ]
    body = _argmin_loss_body
    if with_quant:
        out_specs.append(
            pl.BlockSpec((_BPP, T, C), lambda b, boff=boff: (b + boff, 0, 0)))
        out_shape.append(jax.ShapeDtypeStruct((B, T, C), jnp.float32))
        body = functools.partial(_argmin_loss_body)
    return pl.pallas_call(
        body,
        grid=(grid,),
        in_specs=[
            pl.BlockSpec((_BPP, T, C), lambda b, boff=boff: (b + boff, 0, 0)),
            pl.BlockSpec((K, C), lambda b: (0, 0)),
            pl.BlockSpec((_BPP, T, 1), lambda b, boff=boff: (b + boff, 0, 0)),
            pl.BlockSpec((1, K), lambda b: (0, 0)),
        ],
        out_specs=out_specs,
        out_shape=out_shape,
    )(x, W, x2, w2)


@functools.cache
def _make_sc_gather(V, D, B, OUT_ROWS):
    info = plsc.get_sparse_core_info()
    NC, NS = info.num_cores, info.num_subcores
    NW = NC * NS
    assert B % (8 * NW) == 0
    b_per_w = B // NW
    NCH = 3
    CH = b_per_w // NCH
    assert CH % 8 == 0
    mesh = plsc.VectorSubcoreMesh(core_axis_name="c", subcore_axis_name="s")

    @functools.partial(
        pl.kernel, mesh=mesh,
        out_type=jax.ShapeDtypeStruct((OUT_ROWS, D), jnp.float32),
        scratch_types=[
            pltpu.VMEM((b_per_w,), jnp.int32),
            pltpu.VMEM((NCH, CH, D), jnp.float32),
            [pltpu.SemaphoreType.DMA] * NCH,
            pltpu.SemaphoreType.DMA,
        ],
    )
    def gather(table_hbm, idx_hbm, out_hbm, idx_v, rows_v, gsems, wsem):
        wid = lax.axis_index("s") * NC + lax.axis_index("c")
        base = wid * b_per_w
        pltpu.sync_copy(idx_hbm.at[pl.ds(base, b_per_w)], idx_v)
        hs = [pltpu.async_copy(table_hbm.at[idx_v.at[pl.ds(c * CH, CH)]],
                               rows_v.at[c], gsems[c])
              for c in range(NCH)]
        ws = []
        for c in range(NCH):
            hs[c].wait()
            ws.append(pltpu.async_copy(
                rows_v.at[c], out_hbm.at[pl.ds(base + c * CH, CH)], wsem))
        for w in ws:
            w.wait()

    return gather


def kernel(x, W):
    B, T, C = x.shape
    K = W.shape[0]
    H = B // 2
    x2 = jnp.sum(x * x, axis=-1, keepdims=True)
    w2 = jnp.sum(W * W, axis=-1)[None, :]
    idx_a, loss_a = _argmin_and_loss(x, W, x2, w2, 0, H)
    quant_a = _make_sc_gather(K, C, H * T, H * T)(W, idx_a)
    idx_b, loss_b, quant_bf = _argmin_and_loss(x, W, x2, w2, H, B - H,
                                               with_quant=True)
    quant = lax.dynamic_update_slice(
        quant_bf.reshape(B * T, C), quant_a, (0, 0)).reshape(B, T, C)
    idx = jnp.concatenate([idx_a, idx_b]).reshape(B, T)
    loss = jnp.concatenate([loss_a, loss_b])
    return quant, loss, loss, idx


# speedup vs baseline: 1.2333x; 1.0588x over previous
import functools

import jax
import jax.numpy as jnp
from jax import lax
from jax.experimental import pallas as pl
from jax.experimental.pallas import tpu as pltpu
from jax.experimental.pallas import tpu_sc as plsc

_EPS = 1e-5
_BPP = 2


def _argmin_loss_body(x_ref, w_ref, xs_ref, w2_ref, idx_ref, loss_ref,
                      q_ref=None, *, boff=0):
    nb, T, C = x_ref.shape
    x = x_ref[...].reshape(nb * T, C)
    w = w_ref[...]
    K = w.shape[0]
    R = nb * T

    s = lax.dot_general(x, w, (((1,), (1,)), ((), ())),
                        preferred_element_type=jnp.float32)
    p = pl.program_id(0)
    xs = xs_ref[pl.ds((p + boff) * R, R)].reshape(R, 1)
    w2 = w2_ref[...]
    d2 = (xs + w2) - 2.0 * s

    dmin = jnp.min(d2, axis=1, keepdims=True)
    eqm = d2 == dmin
    kiota = lax.broadcasted_iota(jnp.int32, (R, K), 1).astype(jnp.float32)
    idx = jnp.min(jnp.where(eqm, kiota, float(K)), axis=1,
                  keepdims=True)
    idx_ref[pl.ds(p * R, R)] = idx[:, 0].astype(jnp.int32)

    oh = kiota == idx
    ef = oh.astype(jnp.float32)
    w2_win = lax.dot_general(ef, w2, (((1,), (1,)), ((), ())),
                             preferred_element_type=jnp.float32)
    s_win = 0.5 * ((xs + w2_win) - dmin)
    xn = jnp.maximum(jnp.sqrt(xs), _EPS)
    wn = jnp.maximum(jnp.sqrt(w2_win), _EPS)
    row = (w2_win / (wn * wn) + xs / (xn * xn)
           - 2.0 * s_win / (wn * xn))
    row2 = row.reshape(nb, T)
    for j in range(nb):
        loss_ref[p * nb + j] = jnp.sum(row2[j]) / (T * C)

    if q_ref is not None:
        quant = lax.dot_general(ef, w, (((1,), (0,)), ((), ())),
                                preferred_element_type=jnp.float32)
        q_ref[...] = quant.reshape(nb, T, C)


def _argmin_and_loss(x, W, x2, w2, off, nb, with_quant=False):
    B, T, C = x.shape
    K = W.shape[0]
    grid = nb // _BPP
    boff = off // _BPP
    assert off % _BPP == 0
    out_specs = [
        pl.BlockSpec((nb * T,), lambda b: (0,)),
        pl.BlockSpec(memory_space=pltpu.SMEM),
    ]
    out_shape = [
        jax.ShapeDtypeStruct((nb * T,), jnp.int32),
        jax.ShapeDtypeStruct((nb,), jnp.float32),
    ]
    if with_quant:
        out_specs.append(
            pl.BlockSpec((_BPP, T, C), lambda b, boff=boff: (b + boff, 0, 0)))
        out_shape.append(jax.ShapeDtypeStruct((B, T, C), jnp.float32))
    return pl.pallas_call(
        functools.partial(_argmin_loss_body, boff=boff),
        grid=(grid,),
        in_specs=[
            pl.BlockSpec((_BPP, T, C), lambda b, boff=boff: (b + boff, 0, 0)),
            pl.BlockSpec((K, C), lambda b: (0, 0)),
            pl.BlockSpec((B * T,), lambda b: (0,)),
            pl.BlockSpec((1, K), lambda b: (0, 0)),
        ],
        out_specs=out_specs,
        out_shape=out_shape,
    )(x, W, x2, w2)


@functools.cache
def _make_sc_gather(V, D, B, OUT_ROWS):
    info = plsc.get_sparse_core_info()
    NC, NS = info.num_cores, info.num_subcores
    NW = NC * NS
    assert B % (8 * NW) == 0
    b_per_w = B // NW
    NCH = 3
    CH = b_per_w // NCH
    assert CH % 8 == 0
    mesh = plsc.VectorSubcoreMesh(core_axis_name="c", subcore_axis_name="s")

    @functools.partial(
        pl.kernel, mesh=mesh,
        out_type=jax.ShapeDtypeStruct((OUT_ROWS, D), jnp.float32),
        scratch_types=[
            pltpu.VMEM((b_per_w,), jnp.int32),
            pltpu.VMEM((NCH, CH, D), jnp.float32),
            [pltpu.SemaphoreType.DMA] * NCH,
            pltpu.SemaphoreType.DMA,
        ],
    )
    def gather(table_hbm, idx_hbm, out_hbm, idx_v, rows_v, gsems, wsem):
        wid = lax.axis_index("s") * NC + lax.axis_index("c")
        base = wid * b_per_w
        pltpu.sync_copy(idx_hbm.at[pl.ds(base, b_per_w)], idx_v)
        hs = [pltpu.async_copy(table_hbm.at[idx_v.at[pl.ds(c * CH, CH)]],
                               rows_v.at[c], gsems[c])
              for c in range(NCH)]
        ws = []
        for c in range(NCH):
            hs[c].wait()
            ws.append(pltpu.async_copy(
                rows_v.at[c], out_hbm.at[pl.ds(base + c * CH, CH)], wsem))
        for w in ws:
            w.wait()

    return gather


def kernel(x, W):
    B, T, C = x.shape
    K = W.shape[0]
    H = B // 2
    x2 = jnp.sum(x * x, axis=-1).reshape(B * T)
    w2 = jnp.sum(W * W, axis=-1)[None, :]
    idx_a, loss_a = _argmin_and_loss(x, W, x2, w2, 0, H)
    quant_a = _make_sc_gather(K, C, H * T, H * T)(W, idx_a)
    idx_b, loss_b, quant_bf = _argmin_and_loss(x, W, x2, w2, H, B - H,
                                               with_quant=True)
    quant = lax.dynamic_update_slice(
        quant_bf.reshape(B * T, C), quant_a, (0, 0)).reshape(B, T, C)
    idx = jnp.concatenate([idx_a, idx_b]).reshape(B, T)
    loss = jnp.concatenate([loss_a, loss_b])
    return quant, loss, loss, idx
